# baseline (speedup 1.0000x reference)
import functools

import jax
import jax.numpy as jnp
from jax import lax
from jax.experimental import pallas as pl
from jax.experimental.pallas import tpu as pltpu
from jax.experimental.pallas import tpu_sc as plsc

N = 10000
E = 160000
F_IN = 500
H = 128
C = 3

KP = 512
CP = 8
NC = 2
NS = 16
NW = NC * NS
CHUNK = 128
EW = 5120
EP = NW * EW
RPT = 640
NP = NS * RPT
MB = 1024
EC = 4096
NEB = EP // EC
CH = NP // 128


def _cnt_body(dst_ref, o_ref):
    i = pl.program_id(0)
    d = dst_ref[0]
    hi = d >> 7
    lo = d & 127
    a = (lax.broadcasted_iota(jnp.int32, (CH, EC), 0) == hi).astype(jnp.float32)
    b = (lax.broadcasted_iota(jnp.int32, (128, EC), 0) == lo).astype(jnp.float32)
    part = lax.dot_general(a, b, (((1,), (1,)), ((), ())),
                           preferred_element_type=jnp.float32)

    @pl.when(i == 0)
    def _():
        o_ref[...] = jnp.zeros_like(o_ref)

    o_ref[...] += part


def _tc_cnt(dst3):
    return pl.pallas_call(
        _cnt_body,
        grid=(NEB,),
        in_specs=[pl.BlockSpec((1, 1, EC), lambda i: (i, 0, 0))],
        out_specs=pl.BlockSpec((CH, 128), lambda i: (0, 0)),
        out_shape=jax.ShapeDtypeStruct((CH, 128), jnp.float32),
    )(dst3)


def _tc1_body(x_ref, wl_ref, wr_ref, p_ref, r_ref):
    xb = x_ref[...]
    p_ref[...] = jnp.dot(xb, wl_ref[...], preferred_element_type=jnp.float32)
    r_ref[...] = jnp.dot(xb, wr_ref[...], preferred_element_type=jnp.float32)


def _tc1(x_pad, w1l_t, w1r_t):
    return pl.pallas_call(
        _tc1_body,
        grid=(NP // MB,),
        in_specs=[
            pl.BlockSpec((MB, KP), lambda i: (i, 0)),
            pl.BlockSpec((KP, H), lambda i: (0, 0)),
            pl.BlockSpec((KP, H), lambda i: (0, 0)),
        ],
        out_specs=[
            pl.BlockSpec((MB, H), lambda i: (i, 0)),
            pl.BlockSpec((MB, H), lambda i: (i, 0)),
        ],
        out_shape=[
            jax.ShapeDtypeStruct((NP, H), jnp.float32),
            jax.ShapeDtypeStruct((NP, H), jnp.float32),
        ],
    )(x_pad, w1l_t, w1r_t)


def _sc_agg_body(tab_hbm, src_hbm, dst_hbm, zf_hbm,
                 acc_out0, acc_out1,
                 acc_sh, src_a, dst_a, rows0, rows1, sem0, sem1):
    c = lax.axis_index("c")
    s = lax.axis_index("s")
    wid = s * NC + c
    r0 = s * RPT
    nch = EW // CHUNK
    pltpu.sync_copy(zf_hbm.at[pl.ds(r0, RPT)], acc_sh.at[pl.ds(r0, RPT)])
    pltpu.sync_copy(src_hbm.at[pl.ds(wid * nch, nch)], src_a)
    pltpu.sync_copy(dst_hbm.at[pl.ds(wid * nch, nch)], dst_a)
    plsc.subcore_barrier()

    cp0 = pltpu.async_copy(tab_hbm.at[src_a.at[0]], rows0, sem0)

    def body(g, carry):
        i0 = g * 2
        pltpu.async_copy(tab_hbm.at[src_a.at[i0 + 1]], rows1, sem1)
        pltpu.make_async_copy(tab_hbm.at[src_a.at[i0]], rows0, sem0).wait()
        pltpu.sync_copy(rows0, acc_sh.at[dst_a.at[i0]], add=True)

        @pl.when(g < (nch // 2) - 1)
        def _():
            pltpu.async_copy(tab_hbm.at[src_a.at[i0 + 2]], rows0, sem0)

        pltpu.make_async_copy(tab_hbm.at[src_a.at[i0 + 1]], rows1, sem1).wait()
        pltpu.sync_copy(rows1, acc_sh.at[dst_a.at[i0 + 1]], add=True)
        return carry

    lax.fori_loop(0, nch // 2, body, 0)
    plsc.subcore_barrier()

    @pl.when(c == 0)
    def _():
        pltpu.sync_copy(acc_sh.at[pl.ds(r0, RPT)], acc_out0.at[pl.ds(r0, RPT)])

    @pl.when(c == 1)
    def _():
        pltpu.sync_copy(acc_sh.at[pl.ds(r0, RPT)], acc_out1.at[pl.ds(r0, RPT)])


def _sc_agg(tab, src2, dst2, zf):
    mesh = plsc.VectorSubcoreMesh(core_axis_name="c", subcore_axis_name="s")
    nch = EW // CHUNK
    kfn = pl.kernel(
        _sc_agg_body,
        mesh=mesh,
        compiler_params=pltpu.CompilerParams(use_tc_tiling_on_sc=True),
        out_type=[
            jax.ShapeDtypeStruct((NP, H), jnp.float32),
            jax.ShapeDtypeStruct((NP, H), jnp.float32),
        ],
        scratch_types=[
            pltpu.VMEM_SHARED((NP, H), jnp.float32),
            pltpu.VMEM((nch, CHUNK), jnp.int32),
            pltpu.VMEM((nch, CHUNK), jnp.int32),
            pltpu.VMEM((CHUNK, H), jnp.float32),
            pltpu.VMEM((CHUNK, H), jnp.float32),
            pltpu.SemaphoreType.DMA,
            pltpu.SemaphoreType.DMA,
        ],
    )
    return kfn(tab, src2, dst2, zf)


def _tc2_body(a0_ref, a1_ref, cnt_ref, r1_ref, b1_ref, wl_ref, wr_ref,
              p2_ref, r2_ref):
    ssum = a0_ref[...] + a1_ref[...]
    rinv = 1.0 / jnp.maximum(cnt_ref[...], 1.0)
    h = jnp.maximum(ssum * rinv + b1_ref[...] + r1_ref[...], 0.0)
    p2_ref[...] = jnp.dot(h, wl_ref[...], preferred_element_type=jnp.float32)
    r2_ref[...] = jnp.dot(h, wr_ref[...], preferred_element_type=jnp.float32)


def _tc2(a0, a1, cnt_col, r1, b1r, w2l_t, w2r_t):
    return pl.pallas_call(
        _tc2_body,
        grid=(NP // MB,),
        in_specs=[
            pl.BlockSpec((MB, H), lambda i: (i, 0)),
            pl.BlockSpec((MB, H), lambda i: (i, 0)),
            pl.BlockSpec((MB, 1), lambda i: (i, 0)),
            pl.BlockSpec((MB, H), lambda i: (i, 0)),
            pl.BlockSpec((1, H), lambda i: (0, 0)),
            pl.BlockSpec((H, H), lambda i: (0, 0)),
            pl.BlockSpec((H, CP), lambda i: (0, 0)),
        ],
        out_specs=[
            pl.BlockSpec((MB, H), lambda i: (i, 0)),
            pl.BlockSpec((MB, CP), lambda i: (i, 0)),
        ],
        out_shape=[
            jax.ShapeDtypeStruct((NP, H), jnp.float32),
            jax.ShapeDtypeStruct((NP, CP), jnp.float32),
        ],
    )(a0, a1, cnt_col, r1, b1r, w2l_t, w2r_t)


def _tc3_body(a0_ref, a1_ref, cnt_ref, r2_ref, b2_ref, o_ref):
    ssum = a0_ref[...][:, :CP] + a1_ref[...][:, :CP]
    rinv = 1.0 / jnp.maximum(cnt_ref[...], 1.0)
    o_ref[...] = ssum * rinv + b2_ref[...] + r2_ref[...]


def _tc3(a0, a1, cnt_col, r2, b2r):
    return pl.pallas_call(
        _tc3_body,
        grid=(NP // MB,),
        in_specs=[
            pl.BlockSpec((MB, H), lambda i: (i, 0)),
            pl.BlockSpec((MB, H), lambda i: (i, 0)),
            pl.BlockSpec((MB, 1), lambda i: (i, 0)),
            pl.BlockSpec((MB, CP), lambda i: (i, 0)),
            pl.BlockSpec((1, CP), lambda i: (0, 0)),
        ],
        out_specs=pl.BlockSpec((MB, CP), lambda i: (i, 0)),
        out_shape=jax.ShapeDtypeStruct((NP, CP), jnp.float32),
    )(a0, a1, cnt_col, r2, b2r)


def kernel(x, edge_index, W1l, b1, W1r, W2l, b2, W2r):
    x_pad = jnp.pad(x, ((0, NP - N), (0, KP - F_IN)))
    w1l_t = jnp.pad(W1l.T, ((0, KP - F_IN), (0, 0)))
    w1r_t = jnp.pad(W1r.T, ((0, KP - F_IN), (0, 0)))
    w2l_t = jnp.pad(W2l.T, ((0, 0), (0, H - C)))
    w2r_t = jnp.pad(W2r.T, ((0, 0), (0, CP - C)))
    b1r = b1.reshape(1, H)
    b2r = jnp.pad(b2, (0, CP - C)).reshape(1, CP)

    src = edge_index[0]
    dst = edge_index[1]
    src_p = jnp.concatenate([src, jnp.zeros((EP - E,), jnp.int32)])
    dst_p = jnp.concatenate([dst, jnp.full((EP - E,), N, jnp.int32)])
    dst3 = dst_p.reshape(NEB, 1, EC)
    src2 = src_p.reshape(EP // CHUNK, CHUNK)
    dst2 = dst_p.reshape(EP // CHUNK, CHUNK)
    zf = jnp.zeros((NP, H), jnp.float32)

    cmat = _tc_cnt(dst3)
    cnt_col = cmat.reshape(NP, 1)

    p1, r1 = _tc1(x_pad, w1l_t, w1r_t)
    acc0, acc1 = _sc_agg(p1, src2, dst2, zf)
    p2, r2 = _tc2(acc0, acc1, cnt_col, r1, b1r, w2l_t, w2r_t)

    acc20, acc21 = _sc_agg(p2, src2, dst2, zf)
    out = _tc3(acc20, acc21, cnt_col, r2, b2r)
    return out[:N, :C]


# speedup vs baseline: 1.0003x; 1.0003x over previous
import functools

import jax
import jax.numpy as jnp
from jax import lax
from jax.experimental import pallas as pl
from jax.experimental.pallas import tpu as pltpu
from jax.experimental.pallas import tpu_sc as plsc

N = 10000
E = 160000
F_IN = 500
H = 128
C = 3

KP = 512
CP = 8
NC = 2
NS = 16
NW = NC * NS
CHUNK = 128
EW = 5120
EP = NW * EW
RPT = 640
NP = NS * RPT
MB = 1024
EC = 4096
NEB = EP // EC
CH = NP // 128


def _cnt_body(dst_ref, o_ref):
    i = pl.program_id(0)
    d = dst_ref[0]
    hi = d >> 7
    lo = d & 127
    a = (lax.broadcasted_iota(jnp.int32, (CH, EC), 0) == hi).astype(jnp.float32)
    b = (lax.broadcasted_iota(jnp.int32, (128, EC), 0) == lo).astype(jnp.float32)
    part = lax.dot_general(a, b, (((1,), (1,)), ((), ())),
                           preferred_element_type=jnp.float32)

    @pl.when(i == 0)
    def _():
        o_ref[...] = jnp.zeros_like(o_ref)

    o_ref[...] += part


def _tc_cnt(dst3):
    return pl.pallas_call(
        _cnt_body,
        grid=(NEB,),
        in_specs=[pl.BlockSpec((1, 1, EC), lambda i: (i, 0, 0))],
        out_specs=pl.BlockSpec((CH, 128), lambda i: (0, 0)),
        out_shape=jax.ShapeDtypeStruct((CH, 128), jnp.float32),
    )(dst3)


def _tc1_body(x_ref, wl_ref, wr_ref, p_ref, r_ref):
    xb = x_ref[...]
    p_ref[...] = jnp.dot(xb, wl_ref[...], preferred_element_type=jnp.float32)
    r_ref[...] = jnp.dot(xb, wr_ref[...], preferred_element_type=jnp.float32)


def _tc1(x_pad, w1l_t, w1r_t):
    return pl.pallas_call(
        _tc1_body,
        grid=(NP // MB,),
        in_specs=[
            pl.BlockSpec((MB, KP), lambda i: (i, 0)),
            pl.BlockSpec((KP, H), lambda i: (0, 0)),
            pl.BlockSpec((KP, H), lambda i: (0, 0)),
        ],
        out_specs=[
            pl.BlockSpec((MB, H), lambda i: (i, 0)),
            pl.BlockSpec((MB, H), lambda i: (i, 0)),
        ],
        out_shape=[
            jax.ShapeDtypeStruct((NP, H), jnp.float32),
            jax.ShapeDtypeStruct((NP, H), jnp.float32),
        ],
    )(x_pad, w1l_t, w1r_t)


def _sc_agg_body(tab_hbm, src_hbm, dst_hbm, zf_hbm,
                 acc_out0, acc_out1,
                 acc_sh, src_a, dst_a, rows0, rows1, sem0, sem1):
    c = lax.axis_index("c")
    s = lax.axis_index("s")
    wid = s * NC + c
    r0 = s * RPT
    nch = EW // CHUNK
    pltpu.sync_copy(zf_hbm.at[pl.ds(r0, RPT)], acc_sh.at[pl.ds(r0, RPT)])
    pltpu.sync_copy(src_hbm.at[pl.ds(wid * nch, nch)], src_a)
    pltpu.sync_copy(dst_hbm.at[pl.ds(wid * nch, nch)], dst_a)
    plsc.subcore_barrier()

    cp0 = pltpu.async_copy(tab_hbm.at[src_a.at[0]], rows0, sem0)

    def body(g, carry):
        i0 = g * 2
        pltpu.async_copy(tab_hbm.at[src_a.at[i0 + 1]], rows1, sem1)
        pltpu.make_async_copy(tab_hbm.at[src_a.at[i0]], rows0, sem0).wait()
        pltpu.sync_copy(rows0, acc_sh.at[dst_a.at[i0]], add=True)

        @pl.when(g < (nch // 2) - 1)
        def _():
            pltpu.async_copy(tab_hbm.at[src_a.at[i0 + 2]], rows0, sem0)

        pltpu.make_async_copy(tab_hbm.at[src_a.at[i0 + 1]], rows1, sem1).wait()
        pltpu.sync_copy(rows1, acc_sh.at[dst_a.at[i0 + 1]], add=True)
        return carry

    lax.fori_loop(0, nch // 2, body, 0)
    plsc.subcore_barrier()

    @pl.when(c == 0)
    def _():
        pltpu.sync_copy(acc_sh.at[pl.ds(r0, RPT)], acc_out0.at[pl.ds(r0, RPT)])

    @pl.when(c == 1)
    def _():
        pltpu.sync_copy(acc_sh.at[pl.ds(r0, RPT)], acc_out1.at[pl.ds(r0, RPT)])


def _sc_agg(tab, src2, dst2, zf):
    mesh = plsc.VectorSubcoreMesh(core_axis_name="c", subcore_axis_name="s")
    nch = EW // CHUNK
    kfn = pl.kernel(
        _sc_agg_body,
        mesh=mesh,
        out_type=[
            jax.ShapeDtypeStruct((NP, H), jnp.float32),
            jax.ShapeDtypeStruct((NP, H), jnp.float32),
        ],
        scratch_types=[
            pltpu.VMEM_SHARED((NP, H), jnp.float32),
            pltpu.VMEM((nch, CHUNK), jnp.int32),
            pltpu.VMEM((nch, CHUNK), jnp.int32),
            pltpu.VMEM((CHUNK, H), jnp.float32),
            pltpu.VMEM((CHUNK, H), jnp.float32),
            pltpu.SemaphoreType.DMA,
            pltpu.SemaphoreType.DMA,
        ],
    )
    return kfn(tab, src2, dst2, zf)


def _tc2_body(a0_ref, a1_ref, cnt_ref, r1_ref, b1_ref, wl_ref, wr_ref,
              p2_ref, r2_ref):
    ssum = a0_ref[...] + a1_ref[...]
    rinv = 1.0 / jnp.maximum(cnt_ref[...], 1.0)
    h = jnp.maximum(ssum * rinv + b1_ref[...] + r1_ref[...], 0.0)
    p2_ref[...] = jnp.dot(h, wl_ref[...], preferred_element_type=jnp.float32)
    r2_ref[...] = jnp.dot(h, wr_ref[...], preferred_element_type=jnp.float32)


def _tc2(a0, a1, cnt_col, r1, b1r, w2l_t, w2r_t):
    return pl.pallas_call(
        _tc2_body,
        grid=(NP // MB,),
        in_specs=[
            pl.BlockSpec((MB, H), lambda i: (i, 0)),
            pl.BlockSpec((MB, H), lambda i: (i, 0)),
            pl.BlockSpec((MB, 1), lambda i: (i, 0)),
            pl.BlockSpec((MB, H), lambda i: (i, 0)),
            pl.BlockSpec((1, H), lambda i: (0, 0)),
            pl.BlockSpec((H, H), lambda i: (0, 0)),
            pl.BlockSpec((H, CP), lambda i: (0, 0)),
        ],
        out_specs=[
            pl.BlockSpec((MB, H), lambda i: (i, 0)),
            pl.BlockSpec((MB, CP), lambda i: (i, 0)),
        ],
        out_shape=[
            jax.ShapeDtypeStruct((NP, H), jnp.float32),
            jax.ShapeDtypeStruct((NP, CP), jnp.float32),
        ],
    )(a0, a1, cnt_col, r1, b1r, w2l_t, w2r_t)


def _tc3_body(a0_ref, a1_ref, cnt_ref, r2_ref, b2_ref, o_ref):
    ssum = a0_ref[...][:, :CP] + a1_ref[...][:, :CP]
    rinv = 1.0 / jnp.maximum(cnt_ref[...], 1.0)
    o_ref[...] = ssum * rinv + b2_ref[...] + r2_ref[...]


def _tc3(a0, a1, cnt_col, r2, b2r):
    return pl.pallas_call(
        _tc3_body,
        grid=(NP // MB,),
        in_specs=[
            pl.BlockSpec((MB, H), lambda i: (i, 0)),
            pl.BlockSpec((MB, H), lambda i: (i, 0)),
            pl.BlockSpec((MB, 1), lambda i: (i, 0)),
            pl.BlockSpec((MB, CP), lambda i: (i, 0)),
            pl.BlockSpec((1, CP), lambda i: (0, 0)),
        ],
        out_specs=pl.BlockSpec((MB, CP), lambda i: (i, 0)),
        out_shape=jax.ShapeDtypeStruct((NP, CP), jnp.float32),
    )(a0, a1, cnt_col, r2, b2r)


def kernel(x, edge_index, W1l, b1, W1r, W2l, b2, W2r):
    x_pad = jnp.pad(x, ((0, NP - N), (0, KP - F_IN)))
    w1l_t = jnp.pad(W1l.T, ((0, KP - F_IN), (0, 0)))
    w1r_t = jnp.pad(W1r.T, ((0, KP - F_IN), (0, 0)))
    w2l_t = jnp.pad(W2l.T, ((0, 0), (0, H - C)))
    w2r_t = jnp.pad(W2r.T, ((0, 0), (0, CP - C)))
    b1r = b1.reshape(1, H)
    b2r = jnp.pad(b2, (0, CP - C)).reshape(1, CP)

    src = edge_index[0]
    dst = edge_index[1]
    src_p = jnp.concatenate([src, jnp.zeros((EP - E,), jnp.int32)])
    dst_p = jnp.concatenate([dst, jnp.full((EP - E,), N, jnp.int32)])
    dst3 = dst_p.reshape(NEB, 1, EC)
    src2 = src_p.reshape(EP // CHUNK, CHUNK)
    dst2 = dst_p.reshape(EP // CHUNK, CHUNK)
    zf = jnp.zeros((NP, H), jnp.float32)

    cmat = _tc_cnt(dst3)
    cnt_col = cmat.reshape(NP, 1)

    p1, r1 = _tc1(x_pad, w1l_t, w1r_t)
    acc0, acc1 = _sc_agg(p1, src2, dst2, zf)
    p2, r2 = _tc2(acc0, acc1, cnt_col, r1, b1r, w2l_t, w2r_t)

    acc20, acc21 = _sc_agg(p2, src2, dst2, zf)
    out = _tc3(acc20, acc21, cnt_col, r2, b2r)
    return out[:N, :C]


# speedup vs baseline: 1.1973x; 1.1970x over previous
import functools

import jax
import jax.numpy as jnp
from jax import lax
from jax.experimental import pallas as pl
from jax.experimental.pallas import tpu as pltpu
from jax.experimental.pallas import tpu_sc as plsc

N = 10000
E = 160000
F_IN = 500
H = 128
C = 3

KP = 512
CP = 8
NC = 2
NS = 16
NW = NC * NS
CHUNK = 128
EW = 5120
EP = NW * EW
RPT = 640
NP = NS * RPT
MB = 1024
EC = 4096
NEB = EP // EC
CH = NP // 128


def _cnt_body(dst_ref, o_ref):
    i = pl.program_id(0)
    d = dst_ref[0]
    hi = d >> 7
    lo = d & 127
    a = (lax.broadcasted_iota(jnp.int32, (CH, EC), 0) == hi).astype(jnp.float32)
    b = (lax.broadcasted_iota(jnp.int32, (128, EC), 0) == lo).astype(jnp.float32)
    part = lax.dot_general(a, b, (((1,), (1,)), ((), ())),
                           preferred_element_type=jnp.float32)

    @pl.when(i == 0)
    def _():
        o_ref[...] = jnp.zeros_like(o_ref)

    o_ref[...] += part


def _tc_cnt(dst3):
    return pl.pallas_call(
        _cnt_body,
        grid=(NEB,),
        in_specs=[pl.BlockSpec((1, 1, EC), lambda i: (i, 0, 0))],
        out_specs=pl.BlockSpec((CH, 128), lambda i: (0, 0)),
        out_shape=jax.ShapeDtypeStruct((CH, 128), jnp.float32),
    )(dst3)


def _tc1_body(x_ref, wl_ref, wr_ref, p_ref, r_ref):
    xb = x_ref[...]
    cdims = (((0,), (0,)), ((), ()))
    p_ref[...] = lax.dot_general(xb, wl_ref[...], cdims,
                                 preferred_element_type=jnp.float32)
    r_ref[...] = lax.dot_general(xb, wr_ref[...], cdims,
                                 preferred_element_type=jnp.float32)


def _tc1(xt_pad, w1l_t, w1r_t):
    return pl.pallas_call(
        _tc1_body,
        grid=(NP // MB,),
        in_specs=[
            pl.BlockSpec((KP, MB), lambda i: (0, i)),
            pl.BlockSpec((KP, H), lambda i: (0, 0)),
            pl.BlockSpec((KP, H), lambda i: (0, 0)),
        ],
        out_specs=[
            pl.BlockSpec((MB, H), lambda i: (i, 0)),
            pl.BlockSpec((MB, H), lambda i: (i, 0)),
        ],
        out_shape=[
            jax.ShapeDtypeStruct((NP, H), jnp.float32),
            jax.ShapeDtypeStruct((NP, H), jnp.float32),
        ],
    )(xt_pad, w1l_t, w1r_t)


def _sc_agg_body(tab_hbm, src_hbm, dst_hbm, zf_hbm,
                 acc_out0, acc_out1,
                 acc_sh, src_a, dst_a, rows0, rows1, sem0, sem1):
    c = lax.axis_index("c")
    s = lax.axis_index("s")
    wid = s * NC + c
    r0 = s * RPT
    nch = EW // CHUNK
    pltpu.sync_copy(zf_hbm.at[pl.ds(r0, RPT)], acc_sh.at[pl.ds(r0, RPT)])
    pltpu.sync_copy(src_hbm.at[pl.ds(wid * nch, nch)], src_a)
    pltpu.sync_copy(dst_hbm.at[pl.ds(wid * nch, nch)], dst_a)
    plsc.subcore_barrier()

    cp0 = pltpu.async_copy(tab_hbm.at[src_a.at[0]], rows0, sem0)

    def body(g, carry):
        i0 = g * 2
        pltpu.async_copy(tab_hbm.at[src_a.at[i0 + 1]], rows1, sem1)
        pltpu.make_async_copy(tab_hbm.at[src_a.at[i0]], rows0, sem0).wait()
        pltpu.sync_copy(rows0, acc_sh.at[dst_a.at[i0]], add=True)

        @pl.when(g < (nch // 2) - 1)
        def _():
            pltpu.async_copy(tab_hbm.at[src_a.at[i0 + 2]], rows0, sem0)

        pltpu.make_async_copy(tab_hbm.at[src_a.at[i0 + 1]], rows1, sem1).wait()
        pltpu.sync_copy(rows1, acc_sh.at[dst_a.at[i0 + 1]], add=True)
        return carry

    lax.fori_loop(0, nch // 2, body, 0)
    plsc.subcore_barrier()

    @pl.when(c == 0)
    def _():
        pltpu.sync_copy(acc_sh.at[pl.ds(r0, RPT)], acc_out0.at[pl.ds(r0, RPT)])

    @pl.when(c == 1)
    def _():
        pltpu.sync_copy(acc_sh.at[pl.ds(r0, RPT)], acc_out1.at[pl.ds(r0, RPT)])


def _sc_agg(tab, src2, dst2, zf):
    mesh = plsc.VectorSubcoreMesh(core_axis_name="c", subcore_axis_name="s")
    nch = EW // CHUNK
    kfn = pl.kernel(
        _sc_agg_body,
        mesh=mesh,
        out_type=[
            jax.ShapeDtypeStruct((NP, H), jnp.float32),
            jax.ShapeDtypeStruct((NP, H), jnp.float32),
        ],
        scratch_types=[
            pltpu.VMEM_SHARED((NP, H), jnp.float32),
            pltpu.VMEM((nch, CHUNK), jnp.int32),
            pltpu.VMEM((nch, CHUNK), jnp.int32),
            pltpu.VMEM((CHUNK, H), jnp.float32),
            pltpu.VMEM((CHUNK, H), jnp.float32),
            pltpu.SemaphoreType.DMA,
            pltpu.SemaphoreType.DMA,
        ],
    )
    return kfn(tab, src2, dst2, zf)


def _tc2_body(a0_ref, a1_ref, cnt_ref, r1_ref, b1_ref, wl_ref, wr_ref,
              p2_ref, r2_ref):
    ssum = a0_ref[...] + a1_ref[...]
    rinv = 1.0 / jnp.maximum(cnt_ref[...], 1.0)
    h = jnp.maximum(ssum * rinv + b1_ref[...] + r1_ref[...], 0.0)
    p2_ref[...] = jnp.dot(h, wl_ref[...], preferred_element_type=jnp.float32)
    r2_ref[...] = jnp.dot(h, wr_ref[...], preferred_element_type=jnp.float32)


def _tc2(a0, a1, cnt_col, r1, b1r, w2l_t, w2r_t):
    return pl.pallas_call(
        _tc2_body,
        grid=(NP // MB,),
        in_specs=[
            pl.BlockSpec((MB, H), lambda i: (i, 0)),
            pl.BlockSpec((MB, H), lambda i: (i, 0)),
            pl.BlockSpec((MB, 1), lambda i: (i, 0)),
            pl.BlockSpec((MB, H), lambda i: (i, 0)),
            pl.BlockSpec((1, H), lambda i: (0, 0)),
            pl.BlockSpec((H, H), lambda i: (0, 0)),
            pl.BlockSpec((H, CP), lambda i: (0, 0)),
        ],
        out_specs=[
            pl.BlockSpec((MB, H), lambda i: (i, 0)),
            pl.BlockSpec((MB, CP), lambda i: (i, 0)),
        ],
        out_shape=[
            jax.ShapeDtypeStruct((NP, H), jnp.float32),
            jax.ShapeDtypeStruct((NP, CP), jnp.float32),
        ],
    )(a0, a1, cnt_col, r1, b1r, w2l_t, w2r_t)


def _tc3_body(a0_ref, a1_ref, cnt_ref, r2_ref, b2_ref, o_ref):
    ssum = a0_ref[...][:, :CP] + a1_ref[...][:, :CP]
    rinv = 1.0 / jnp.maximum(cnt_ref[...], 1.0)
    o_ref[...] = ssum * rinv + b2_ref[...] + r2_ref[...]


def _tc3(a0, a1, cnt_col, r2, b2r):
    return pl.pallas_call(
        _tc3_body,
        grid=(NP // MB,),
        in_specs=[
            pl.BlockSpec((MB, H), lambda i: (i, 0)),
            pl.BlockSpec((MB, H), lambda i: (i, 0)),
            pl.BlockSpec((MB, 1), lambda i: (i, 0)),
            pl.BlockSpec((MB, CP), lambda i: (i, 0)),
            pl.BlockSpec((1, CP), lambda i: (0, 0)),
        ],
        out_specs=pl.BlockSpec((MB, CP), lambda i: (i, 0)),
        out_shape=jax.ShapeDtypeStruct((NP, CP), jnp.float32),
    )(a0, a1, cnt_col, r2, b2r)


def kernel(x, edge_index, W1l, b1, W1r, W2l, b2, W2r):
    xt_pad = jnp.pad(x.T, ((0, KP - F_IN), (0, NP - N)))
    w1l_t = jnp.pad(W1l.T, ((0, KP - F_IN), (0, 0)))
    w1r_t = jnp.pad(W1r.T, ((0, KP - F_IN), (0, 0)))
    w2l_t = jnp.pad(W2l.T, ((0, 0), (0, H - C)))
    w2r_t = jnp.pad(W2r.T, ((0, 0), (0, CP - C)))
    b1r = b1.reshape(1, H)
    b2r = jnp.pad(b2, (0, CP - C)).reshape(1, CP)

    src = edge_index[0]
    dst = edge_index[1]
    src_p = jnp.concatenate([src, jnp.zeros((EP - E,), jnp.int32)])
    dst_p = jnp.concatenate([dst, jnp.full((EP - E,), N, jnp.int32)])
    dst3 = dst_p.reshape(NEB, 1, EC)
    src2 = src_p.reshape(EP // CHUNK, CHUNK)
    dst2 = dst_p.reshape(EP // CHUNK, CHUNK)
    zf = jnp.zeros((NP, H), jnp.float32)

    cmat = _tc_cnt(dst3)
    cnt_col = cmat.reshape(NP, 1)

    p1, r1 = _tc1(xt_pad, w1l_t, w1r_t)
    acc0, acc1 = _sc_agg(p1, src2, dst2, zf)
    p2, r2 = _tc2(acc0, acc1, cnt_col, r1, b1r, w2l_t, w2r_t)

    acc20, acc21 = _sc_agg(p2, src2, dst2, zf)
    out = _tc3(acc20, acc21, cnt_col, r2, b2r)
    return out[:N, :C]
